# Initial kernel scaffold; baseline (speedup 1.0000x reference)
import jax
import jax.numpy as jnp
from jax.experimental import pallas as pl


def kernel(z, edge_index, batch, W, b):
    raise NotImplementedError("write your pallas kernel here")


# speedup vs baseline: 4.7453x; 4.7453x over previous
import jax
import jax.numpy as jnp
from jax import lax
from jax.experimental import pallas as pl
from jax.experimental.pallas import tpu as pltpu

_N = 50000
_H = 256
_S = 64
_A = 16
_R = 1000
_NBLK = _N // _R

_NEG = float("-inf")


def _tc_body(bf_ref, z_ref, w_ref, b_ref, out_ref, acc_ref):
    blk = pl.program_id(0)

    @pl.when(blk == 0)
    def _init():
        acc_ref[...] = jnp.full((_S, _H), _NEG, jnp.float32)

    zb = z_ref[...]
    bf = bf_ref[0]
    lo = jnp.min(bf).astype(jnp.int32)
    hi = jnp.max(bf).astype(jnp.int32)

    def seg_body(s, carry):
        mask = bf == s.astype(jnp.float32)
        vals = jnp.where(mask, zb, _NEG)
        m = jnp.max(vals, axis=0, keepdims=True)
        cur = acc_ref[pl.ds(s, 1), :]
        acc_ref[pl.ds(s, 1), :] = jnp.maximum(cur, m)
        return carry

    lax.fori_loop(lo, hi + 1, seg_body, 0)

    @pl.when(blk == _NBLK - 1)
    def _finish():
        pooled = acc_ref[...]
        out = lax.dot_general(
            pooled, w_ref[...], (((1,), (1,)), ((), ())),
            preferred_element_type=jnp.float32)
        out_ref[...] = out + b_ref[...]


def _zero():
    return jnp.zeros((), jnp.int32)


def kernel(z, edge_index, batch, W, b):
    bf = batch.astype(jnp.float32).reshape(_NBLK, _R, 1)
    b2 = b.astype(jnp.float32).reshape(1, _A)
    out = pl.pallas_call(
        _tc_body,
        grid=(_NBLK,),
        in_specs=[
            pl.BlockSpec((1, _R, 1), lambda i: (i, _zero(), _zero())),
            pl.BlockSpec((_R, _H), lambda i: (i, _zero())),
            pl.BlockSpec((_A, _H), lambda i: (_zero(), _zero())),
            pl.BlockSpec((1, _A), lambda i: (_zero(), _zero())),
        ],
        out_specs=pl.BlockSpec((_S, _A), lambda i: (_zero(), _zero())),
        out_shape=jax.ShapeDtypeStruct((_S, _A), jnp.float32),
        scratch_shapes=[pltpu.VMEM((_S, _H), jnp.float32)],
    )(bf, z.astype(jnp.float32), W.astype(jnp.float32), b2)
    return out
